# baseline (speedup 1.0000x reference)
import functools

import jax
import jax.numpy as jnp
from jax import lax
from jax.experimental import pallas as pl
from jax.experimental.pallas import tpu as pltpu
from jax.experimental.pallas import tpu_sc as plsc

_E = 8192
_D = 64
_TB = 256

_NC, _NS = 2, 16
_NW = _NC * _NS
_BPW = _E // _NW
_CHUNK = 128


def _dist_body(zsq_ref, csq_ref, cbt_ref, z_ref, logits_ref, idx_ref):
    z = z_ref[0]
    zsq = zsq_ref[0]
    mm2 = lax.dot_general(z * 2.0, cbt_ref[...], (((1,), (0,)), ((), ())),
                          preferred_element_type=jnp.float32)
    logits = mm2 - (zsq + csq_ref[...])
    logits_ref[...] = logits[None]
    rowmax = jnp.max(logits, axis=1, keepdims=True)
    ii = lax.broadcasted_iota(jnp.int32, logits.shape, 1)
    idxcol = jnp.min(jnp.where(logits == rowmax, ii, jnp.int32(_E)),
                     axis=1, keepdims=True)
    idx_ref[...] = idxcol.reshape(1, _TB // _CHUNK, _CHUNK)


def _tc_distance(z3, cbt, csq, zsq3):
    b, s, _ = z3.shape
    spt = s // _TB
    return pl.pallas_call(
        _dist_body,
        grid=(b * spt,),
        in_specs=[
            pl.BlockSpec((1, _TB, 1), lambda t: (t // 4, t % 4, 0)),
            pl.BlockSpec((1, _E), lambda t: (0, 0)),
            pl.BlockSpec((_D, _E), lambda t: (0, 0)),
            pl.BlockSpec((1, _TB, _D), lambda t: (t // 4, t % 4, 0)),
        ],
        out_specs=[
            pl.BlockSpec((1, _TB, _E), lambda t: (t // 4, t % 4, 0)),
            pl.BlockSpec((1, _TB // _CHUNK, _CHUNK), lambda t: (t, 0, 0)),
        ],
        out_shape=[
            jax.ShapeDtypeStruct((b, s, _E), jnp.float32),
            jax.ShapeDtypeStruct((b * spt, _TB // _CHUNK, _CHUNK),
                                 jnp.int32),
        ],
    )(zsq3, csq, cbt, z3)


_DPAD = 128


def _sc_gather(cb_pad, idx3d):
    mesh = plsc.VectorSubcoreMesh(core_axis_name="c", subcore_axis_name="s")

    @functools.partial(
        pl.kernel, mesh=mesh,
        out_type=jax.ShapeDtypeStruct((_E, _DPAD), jnp.float32),
        scratch_types=[
            pltpu.VMEM((_BPW // _CHUNK, _CHUNK), jnp.int32),
            pltpu.VMEM((_BPW, _DPAD), jnp.float32),
            pltpu.SemaphoreType.DMA,
        ],
    )
    def gather_kernel(cb_hbm, idx_hbm, out_hbm, idx_v, rows_v, sem):
        wid = lax.axis_index("s") * _NC + lax.axis_index("c")
        pltpu.sync_copy(idx_hbm.at[wid], idx_v)
        copies = [
            pltpu.async_copy(cb_hbm.at[idx_v.at[k]],
                             rows_v.at[pl.ds(k * _CHUNK, _CHUNK)], sem)
            for k in range(_BPW // _CHUNK)
        ]
        for c in copies:
            c.wait()
        pltpu.sync_copy(rows_v, out_hbm.at[pl.ds(wid * _BPW, _BPW)])

    return gather_kernel(cb_pad, idx3d)


def kernel(z, codebook):
    b, s, d = z.shape
    zf = z.reshape(-1, d)
    cbt = codebook.T
    csq = jnp.sum(codebook ** 2, axis=1).reshape(1, -1)
    zsq3 = jnp.sum(zf ** 2, axis=1, keepdims=True).reshape(b, s, 1)
    neg_dist, idx3 = _tc_distance(z, cbt, csq, zsq3)
    cb_pad = jnp.pad(codebook, ((0, 0), (0, _DPAD - _D)))
    quantized = _sc_gather(cb_pad, idx3)[:, :_D]
    loss = jnp.zeros((), jnp.float32)
    return (quantized.reshape(b, s, d),
            idx3.reshape(b, s),
            loss,
            neg_dist)


# speedup vs baseline: 1.0737x; 1.0737x over previous
import functools

import jax
import jax.numpy as jnp
from jax import lax
from jax.experimental import pallas as pl
from jax.experimental.pallas import tpu as pltpu
from jax.experimental.pallas import tpu_sc as plsc

_E = 8192
_D = 64
_TB = 256

_NC, _NS = 2, 16
_NW = _NC * _NS
_BPW = _E // _NW
_CHUNK = 128


def _dist_body(zsq_ref, csq_ref, cbt_ref, z_ref, logits_ref, idx_ref):
    z = z_ref[...]
    zsq = zsq_ref[...]
    mm2 = lax.dot_general(z * 2.0, cbt_ref[...], (((1,), (0,)), ((), ())),
                          preferred_element_type=jnp.float32)
    logits = mm2 - (zsq + csq_ref[...])
    logits_ref[...] = logits
    rowmax = jnp.max(logits, axis=1, keepdims=True)
    ii = lax.broadcasted_iota(jnp.int32, logits.shape, 1)
    idxcol = jnp.min(jnp.where(logits == rowmax, ii, jnp.int32(_E)),
                     axis=1, keepdims=True)
    idx_ref[...] = idxcol.reshape(1, _TB // _CHUNK, _CHUNK)


def _tc_distance(z_flat, cbt, csq, zsq):
    n = z_flat.shape[0]
    return pl.pallas_call(
        _dist_body,
        grid=(n // _TB,),
        in_specs=[
            pl.BlockSpec((_TB, 1), lambda t: (t, 0)),
            pl.BlockSpec((1, _E), lambda t: (0, 0)),
            pl.BlockSpec((_D, _E), lambda t: (0, 0)),
            pl.BlockSpec((_TB, _D), lambda t: (t, 0)),
        ],
        out_specs=[
            pl.BlockSpec((_TB, _E), lambda t: (t, 0)),
            pl.BlockSpec((1, _TB // _CHUNK, _CHUNK), lambda t: (t, 0, 0)),
        ],
        out_shape=[
            jax.ShapeDtypeStruct((n, _E), jnp.float32),
            jax.ShapeDtypeStruct((n // _TB, _TB // _CHUNK, _CHUNK),
                                 jnp.int32),
        ],
    )(zsq, csq, cbt, z_flat)


_DPAD = 128


def _sc_gather(cb_pad, idx3d):
    mesh = plsc.VectorSubcoreMesh(core_axis_name="c", subcore_axis_name="s")

    @functools.partial(
        pl.kernel, mesh=mesh,
        out_type=jax.ShapeDtypeStruct((_E, _DPAD), jnp.float32),
        scratch_types=[
            pltpu.VMEM((_BPW // _CHUNK, _CHUNK), jnp.int32),
            pltpu.VMEM((_BPW, _DPAD), jnp.float32),
            pltpu.SemaphoreType.DMA,
        ],
    )
    def gather_kernel(cb_hbm, idx_hbm, out_hbm, idx_v, rows_v, sem):
        wid = lax.axis_index("s") * _NC + lax.axis_index("c")
        pltpu.sync_copy(idx_hbm.at[wid], idx_v)
        copies = [
            pltpu.async_copy(cb_hbm.at[idx_v.at[k]],
                             rows_v.at[pl.ds(k * _CHUNK, _CHUNK)], sem)
            for k in range(_BPW // _CHUNK)
        ]
        for c in copies:
            c.wait()
        pltpu.sync_copy(rows_v, out_hbm.at[pl.ds(wid * _BPW, _BPW)])

    return gather_kernel(cb_pad, idx3d)


def kernel(z, codebook):
    b, s, d = z.shape
    zf = z.reshape(-1, d)
    cbt = codebook.T
    csq = jnp.sum(codebook ** 2, axis=1).reshape(1, -1)
    zsq = jnp.sum(zf ** 2, axis=1, keepdims=True)
    neg_dist, idx3 = _tc_distance(zf, cbt, csq, zsq)
    cb_pad = jnp.pad(codebook, ((0, 0), (0, _DPAD - _D)))
    quantized = _sc_gather(cb_pad, idx3)[:, :_D]
    loss = jnp.zeros((), jnp.float32)
    return (quantized.reshape(b, s, d),
            idx3.reshape(b, s),
            loss,
            neg_dist.reshape(b, s, _E))


# speedup vs baseline: 1.0819x; 1.0076x over previous
import functools

import jax
import jax.numpy as jnp
from jax import lax
from jax.experimental import pallas as pl
from jax.experimental.pallas import tpu as pltpu
from jax.experimental.pallas import tpu_sc as plsc

_E = 8192
_D = 64
_TB = 512

_NC, _NS = 2, 16
_NW = _NC * _NS
_BPW = _E // _NW
_CHUNK = 128


def _dist_body(zsq_ref, csq_ref, cbt_ref, z_ref, logits_ref, idx_ref):
    z = z_ref[...]
    zsq = zsq_ref[...]
    mm2 = lax.dot_general(z * 2.0, cbt_ref[...], (((1,), (0,)), ((), ())),
                          preferred_element_type=jnp.float32)
    logits = mm2 - (zsq + csq_ref[...])
    logits_ref[...] = logits
    rowmax = jnp.max(logits, axis=1, keepdims=True)
    ii = lax.broadcasted_iota(jnp.int32, logits.shape, 1)
    idxcol = jnp.min(jnp.where(logits == rowmax, ii, jnp.int32(_E)),
                     axis=1, keepdims=True)
    idx_ref[...] = idxcol.reshape(1, _TB // _CHUNK, _CHUNK)


def _tc_distance(z_flat, cbt, csq, zsq):
    n = z_flat.shape[0]
    return pl.pallas_call(
        _dist_body,
        grid=(n // _TB,),
        in_specs=[
            pl.BlockSpec((_TB, 1), lambda t: (t, 0)),
            pl.BlockSpec((1, _E), lambda t: (0, 0)),
            pl.BlockSpec((_D, _E), lambda t: (0, 0)),
            pl.BlockSpec((_TB, _D), lambda t: (t, 0)),
        ],
        out_specs=[
            pl.BlockSpec((_TB, _E), lambda t: (t, 0)),
            pl.BlockSpec((1, _TB // _CHUNK, _CHUNK), lambda t: (t, 0, 0)),
        ],
        out_shape=[
            jax.ShapeDtypeStruct((n, _E), jnp.float32),
            jax.ShapeDtypeStruct((n // _TB, _TB // _CHUNK, _CHUNK),
                                 jnp.int32),
        ],
    )(zsq, csq, cbt, z_flat)


_DPAD = 128


def _sc_gather(cb_pad, idx3d):
    mesh = plsc.VectorSubcoreMesh(core_axis_name="c", subcore_axis_name="s")

    @functools.partial(
        pl.kernel, mesh=mesh,
        out_type=jax.ShapeDtypeStruct((_E, _DPAD), jnp.float32),
        scratch_types=[
            pltpu.VMEM((_BPW // _CHUNK, _CHUNK), jnp.int32),
            pltpu.VMEM((_BPW, _DPAD), jnp.float32),
            pltpu.SemaphoreType.DMA,
        ],
    )
    def gather_kernel(cb_hbm, idx_hbm, out_hbm, idx_v, rows_v, sem):
        wid = lax.axis_index("s") * _NC + lax.axis_index("c")
        cpb = _TB // _CHUNK
        c0 = wid * (_BPW // _CHUNK)
        pltpu.sync_copy(idx_hbm.at[c0 // cpb, pl.ds(c0 % cpb,
                                                    _BPW // _CHUNK)], idx_v)
        copies = [
            pltpu.async_copy(cb_hbm.at[idx_v.at[k]],
                             rows_v.at[pl.ds(k * _CHUNK, _CHUNK)], sem)
            for k in range(_BPW // _CHUNK)
        ]
        for c in copies:
            c.wait()
        pltpu.sync_copy(rows_v, out_hbm.at[pl.ds(wid * _BPW, _BPW)])

    return gather_kernel(cb_pad, idx3d)


def kernel(z, codebook):
    b, s, d = z.shape
    zf = z.reshape(-1, d)
    cbt = codebook.T
    csq = jnp.sum(codebook ** 2, axis=1).reshape(1, -1)
    zsq = jnp.sum(zf ** 2, axis=1, keepdims=True)
    neg_dist, idx3 = _tc_distance(zf, cbt, csq, zsq)
    cb_pad = jnp.pad(codebook, ((0, 0), (0, _DPAD - _D)))
    quantized = _sc_gather(cb_pad, idx3)[:, :_D]
    loss = jnp.zeros((), jnp.float32)
    return (quantized.reshape(b, s, d),
            idx3.reshape(b, s),
            loss,
            neg_dist.reshape(b, s, _E))
